# baseline (speedup 1.0000x reference)
import functools

import jax
import jax.numpy as jnp
from jax import lax
from jax.experimental import pallas as pl
from jax.experimental.pallas import tpu as pltpu
from jax.experimental.pallas import tpu_sc as plsc

_INFO = plsc.get_sparse_core_info()
_NC, _NS, _L = _INFO.num_cores, _INFO.num_subcores, _INFO.num_lanes
_NW = _NC * _NS


@functools.lru_cache(maxsize=None)
def _make_sc_gather_sum(batch: int, seq: int, vocab: int):
    n_rows = batch * seq
    per_w = n_rows // _NW
    n_vec = per_w // _L
    mesh = plsc.VectorSubcoreMesh(core_axis_name="c", subcore_axis_name="s")

    @functools.partial(
        pl.kernel,
        mesh=mesh,
        compiler_params=pltpu.CompilerParams(needs_layout_passes=False),
        out_type=jax.ShapeDtypeStruct((_NW, _L), jnp.float32),
        scratch_types=[
            pltpu.VMEM((batch, seq), jnp.int32),
            pltpu.VMEM((_L * 8, 128), jnp.float32),
            pltpu.VMEM((_L * 8, 128), jnp.float32),
            pltpu.VMEM((_L * 8, 128), jnp.float32),
            pltpu.VMEM((_L,), jnp.float32),
            pltpu.SemaphoreType.DMA,
            pltpu.SemaphoreType.DMA,
            pltpu.SemaphoreType.DMA,
        ],
    )
    def sc_kernel(logits_hbm, tid_hbm, out_hbm,
                  tid_v, buf_0, buf_1, buf_2, stage_v,
                  sem_0, sem_1, sem_2):
        cid = lax.axis_index("c")
        sid = lax.axis_index("s")
        wid = sid * _NC + cid
        base = wid * per_w
        b_idx = base // seq

        pltpu.sync_copy(tid_hbm, tid_v)

        nbuf = 3
        bufs = (buf_0, buf_1, buf_2)
        sems = (sem_0, sem_1, sem_2)

        def fire(g):
            s0 = base % seq + g * _L
            vec = tid_v[b_idx, pl.ds(s0, _L)]
            buf, sem = bufs[g % nbuf], sems[g % nbuf]
            handles = []
            for k in range(_L):
                j = g * _L + k
                v = lax.reshape(lax.slice(vec, (k,), (k + 1,)), ())
                c0 = pl.multiple_of(jnp.bitwise_and(v, jnp.int32(-128)), 128)
                t0 = pl.multiple_of((base + j) & jnp.int32(-8), 8)
                handles.append(pltpu.async_copy(
                    logits_hbm.at[pl.ds(t0, 8), pl.ds(c0, 128)],
                    buf.at[pl.ds(k * 8, 8)],
                    sem,
                ))
            return vec, handles

        lane = lax.iota(jnp.int32, _L)
        rows = lane * 8 + (lane & 7)
        acc = jnp.zeros((_L,), jnp.float32)
        pending = [fire(g) for g in range(min(nbuf - 1, n_vec))]
        for g in range(n_vec):
            vec, handles = pending.pop(0)
            nxt = g + nbuf - 1
            if nxt < n_vec:
                pending.append(fire(nxt))
            for h in handles:
                h.wait()
            acc = acc + plsc.load_gather(bufs[g % nbuf], [rows, vec & 127])
        stage_v[...] = acc

        pltpu.sync_copy(stage_v, out_hbm.at[wid])

    return sc_kernel


def kernel(logits, target_ids):
    batch, seq, vocab = logits.shape
    logits2d = logits.reshape((batch * seq, vocab))
    tid = target_ids.astype(jnp.int32)
    partials = _make_sc_gather_sum(batch, seq, vocab)(logits2d, tid)
    return -(jnp.sum(partials) / batch)
